# Initial kernel scaffold; baseline (speedup 1.0000x reference)
import jax
import jax.numpy as jnp
from jax.experimental import pallas as pl


def kernel(inputs, embeddings):
    raise NotImplementedError("write your pallas kernel here")


# speedup vs baseline: 1.0936x; 1.0936x over previous
import functools

import jax
import jax.numpy as jnp
from jax import lax
from jax.experimental import pallas as pl
from jax.experimental.pallas import tpu as pltpu
from jax.experimental.pallas import tpu_sc as plsc

EMBED_D = 32
CHUNK = 1024


@functools.cache
def _make_gather(B: int, V: int, D: int):
    info = plsc.get_sparse_core_info()
    NC, NS = info.num_cores, info.num_subcores
    NW = NC * NS
    assert B % NW == 0
    b_per_w = B // NW
    assert b_per_w % CHUNK == 0
    n_chunks = b_per_w // CHUNK

    mesh = plsc.VectorSubcoreMesh(core_axis_name="c", subcore_axis_name="s")

    @functools.partial(
        pl.kernel,
        mesh=mesh,
        out_type=jax.ShapeDtypeStruct((B, D), jnp.float32),
        scratch_types=[
            pltpu.VMEM((CHUNK,), jnp.int32),
            pltpu.VMEM((CHUNK, D), jnp.float32),
            pltpu.SemaphoreType.DMA,
        ],
        compiler_params=pltpu.CompilerParams(use_tc_tiling_on_sc=False),
    )
    def gather_kernel(table_hbm, idx_hbm, out_hbm, idx_v, rows_v, sem):
        wid = lax.axis_index("s") * NC + lax.axis_index("c")
        base = wid * b_per_w

        def body(c, carry):
            off = base + c * CHUNK
            pltpu.sync_copy(idx_hbm.at[pl.ds(off, CHUNK)], idx_v)
            pltpu.async_copy(table_hbm.at[idx_v], rows_v, sem).wait()
            pltpu.sync_copy(rows_v, out_hbm.at[pl.ds(off, CHUNK)])
            return carry

        lax.fori_loop(0, n_chunks, body, 0)

    return gather_kernel


def kernel(inputs, embeddings):
    in_shape = inputs.shape
    flat_idx = jnp.reshape(inputs, (-1,)).astype(jnp.int32)
    B = flat_idx.shape[0]
    V, D = embeddings.shape
    out = _make_gather(B, V, D)(embeddings, flat_idx)
    return jnp.reshape(out, in_shape + (D,))


# speedup vs baseline: 1.1088x; 1.0139x over previous
import functools

import jax
import jax.numpy as jnp
from jax import lax
from jax.experimental import pallas as pl
from jax.experimental.pallas import tpu as pltpu
from jax.experimental.pallas import tpu_sc as plsc

CHUNK = 1024


@functools.cache
def _make_gather(B: int, V: int, D: int):
    info = plsc.get_sparse_core_info()
    NC, NS = info.num_cores, info.num_subcores
    NW = NC * NS
    assert B % NW == 0
    b_per_w = B // NW
    assert b_per_w % CHUNK == 0
    n_chunks = b_per_w // CHUNK

    mesh = plsc.VectorSubcoreMesh(core_axis_name="c", subcore_axis_name="s")

    @functools.partial(
        pl.kernel,
        mesh=mesh,
        out_type=jax.ShapeDtypeStruct((B, D), jnp.float32),
        scratch_types=[
            pltpu.VMEM((CHUNK,), jnp.int32),
            pltpu.VMEM((CHUNK,), jnp.int32),
            pltpu.VMEM((CHUNK, D), jnp.float32),
            pltpu.VMEM((CHUNK, D), jnp.float32),
            pltpu.SemaphoreType.DMA,
            pltpu.SemaphoreType.DMA,
            pltpu.SemaphoreType.DMA,
            pltpu.SemaphoreType.DMA,
            pltpu.SemaphoreType.DMA,
            pltpu.SemaphoreType.DMA,
        ],
        compiler_params=pltpu.CompilerParams(use_tc_tiling_on_sc=False),
    )
    def gather_kernel(table_hbm, idx_hbm, out_hbm,
                      idx0, idx1, rows0, rows1,
                      si0, si1, sg0, sg1, so0, so1):
        wid = lax.axis_index("s") * NC + lax.axis_index("c")
        base = wid * b_per_w
        idxv, rows = (idx0, idx1), (rows0, rows1)
        si, sg, so = (si0, si1), (sg0, sg1), (so0, so1)

        def start_idx(c):
            off = base + c * CHUNK
            return pltpu.async_copy(
                idx_hbm.at[pl.ds(off, CHUNK)], idxv[c % 2], si[c % 2])

        def start_gather(c):
            return pltpu.async_copy(
                table_hbm.at[idxv[c % 2]], rows[c % 2], sg[c % 2])

        def start_out(c):
            off = base + c * CHUNK
            return pltpu.async_copy(
                rows[c % 2], out_hbm.at[pl.ds(off, CHUNK)], so[c % 2])

        hi, hg, ho = [None, None], [None, None], [None, None]
        hi[0] = start_idx(0)
        if n_chunks > 1:
            hi[1] = start_idx(1)
        hi[0].wait()
        hg[0] = start_gather(0)

        for c in range(n_chunks):
            b = c % 2
            nb = 1 - b
            hg[b].wait()
            ho[b] = start_out(c)
            if c + 2 < n_chunks:
                hi[b] = start_idx(c + 2)
            if c + 1 < n_chunks:
                hi[nb].wait()
                if c >= 1:
                    ho[nb].wait()
                hg[nb] = start_gather(c + 1)

        ho[(n_chunks - 2) % 2].wait()
        ho[(n_chunks - 1) % 2].wait()

    return gather_kernel


def kernel(inputs, embeddings):
    in_shape = inputs.shape
    flat_idx = jnp.reshape(inputs, (-1,)).astype(jnp.int32)
    B = flat_idx.shape[0]
    V, D = embeddings.shape
    out = _make_gather(B, V, D)(embeddings, flat_idx)
    return jnp.reshape(out, in_shape + (D,))


# speedup vs baseline: 1.7760x; 1.6017x over previous
import functools

import jax
import jax.numpy as jnp
from jax import lax
from jax.experimental import pallas as pl
from jax.experimental.pallas import tpu as pltpu
from jax.experimental.pallas import tpu_sc as plsc

CH_S = 32


@functools.cache
def _make_gather(S: int, H: int, V: int, D: int):
    info = plsc.get_sparse_core_info()
    NC, NS = info.num_cores, info.num_subcores
    NW = NC * NS
    assert S % NW == 0
    s_per_w = S // NW
    assert s_per_w % CH_S == 0
    n_chunks = s_per_w // CH_S
    CHUNK = CH_S * H

    mesh = plsc.VectorSubcoreMesh(core_axis_name="c", subcore_axis_name="s")

    @functools.partial(
        pl.kernel,
        mesh=mesh,
        out_type=jax.ShapeDtypeStruct((S, H, D), jnp.float32),
        scratch_types=[
            pltpu.VMEM((CH_S, H), jnp.int32),
            pltpu.VMEM((CH_S, H), jnp.int32),
            pltpu.VMEM((CHUNK,), jnp.int32),
            pltpu.VMEM((CHUNK,), jnp.int32),
            pltpu.VMEM((CHUNK, D), jnp.float32),
            pltpu.VMEM((CHUNK, D), jnp.float32),
            pltpu.SemaphoreType.DMA,
            pltpu.SemaphoreType.DMA,
            pltpu.SemaphoreType.DMA,
            pltpu.SemaphoreType.DMA,
            pltpu.SemaphoreType.DMA,
            pltpu.SemaphoreType.DMA,
        ],
        compiler_params=pltpu.CompilerParams(use_tc_tiling_on_sc=False),
    )
    def gather_kernel(table_hbm, idx2d_hbm, out3d_hbm,
                      ida, idb, fla, flb, rowsa, rowsb,
                      si0, si1, sg0, sg1, so0, so1):
        wid = lax.axis_index("s") * NC + lax.axis_index("c")
        sbase = wid * s_per_w
        idxv, flat, rows = (ida, idb), (fla, flb), (rowsa, rowsb)
        si, sg, so = (si0, si1), (sg0, sg1), (so0, so1)

        cols = []
        col = 0
        while col + 16 <= H:
            cols.append(col)
            col += 16
        if col < H:
            cols.append(H - 16)

        def start_idx(c):
            s0 = sbase + c * CH_S
            return pltpu.async_copy(
                idx2d_hbm.at[pl.ds(s0, CH_S), :], idxv[c % 2], si[c % 2])

        def flatten(c):
            b = c % 2
            src, dst = idxv[b], flat[b]
            for k in range(CH_S):
                for col0 in cols:
                    dst[pl.ds(k * H + col0, 16)] = src[k, pl.ds(col0, 16)]

        def start_gather(c):
            return pltpu.async_copy(
                table_hbm.at[flat[c % 2]], rows[c % 2], sg[c % 2])

        def start_out(c):
            b = c % 2
            s0 = sbase + c * CH_S
            return [
                pltpu.async_copy(
                    rows[b].at[pl.ds(k * H, H), :], out3d_hbm.at[s0 + k], so[b])
                for k in range(CH_S)
            ]

        hi, hg, ho = [None, None], [None, None], [None, None]
        hi[0] = start_idx(0)
        if n_chunks > 1:
            hi[1] = start_idx(1)
        hi[0].wait()
        flatten(0)
        hg[0] = start_gather(0)

        for c in range(n_chunks):
            b = c % 2
            nb = 1 - b
            hg[b].wait()
            ho[b] = start_out(c)
            if c + 2 < n_chunks:
                hi[b] = start_idx(c + 2)
            if c + 1 < n_chunks:
                hi[nb].wait()
                flatten(c + 1)
                if c >= 1:
                    for h in ho[nb]:
                        h.wait()
                hg[nb] = start_gather(c + 1)

        for h in ho[(n_chunks - 2) % 2]:
            h.wait()
        for h in ho[(n_chunks - 1) % 2]:
            h.wait()

    return gather_kernel


def kernel(inputs, embeddings):
    S, H = inputs.shape
    V, D = embeddings.shape
    return _make_gather(S, H, V, D)(embeddings, inputs.astype(jnp.int32))


# speedup vs baseline: 2.6155x; 1.4727x over previous
import functools

import jax
import jax.numpy as jnp
from jax import lax
from jax.experimental import pallas as pl
from jax.experimental.pallas import tpu as pltpu
from jax.experimental.pallas import tpu_sc as plsc

TPAD = 521


@functools.cache
def _make_gather(S: int, H: int, V: int, D: int):
    info = plsc.get_sparse_core_info()
    NC, NS = info.num_cores, info.num_subcores
    NW = NC * NS
    assert S % NW == 0
    SW = S // NW

    mesh = plsc.VectorSubcoreMesh(core_axis_name="c", subcore_axis_name="s")

    @functools.partial(
        pl.kernel,
        mesh=mesh,
        out_type=jax.ShapeDtypeStruct((H, D, S), jnp.float32),
        scratch_types=[
            pltpu.VMEM((SW,), jnp.int32),
            pltpu.VMEM((SW,), jnp.int32),
            pltpu.VMEM((SW, D), jnp.float32),
            pltpu.VMEM((SW, D), jnp.float32),
            pltpu.VMEM((D, TPAD), jnp.float32),
            pltpu.VMEM((D, TPAD), jnp.float32),
            pltpu.SemaphoreType.DMA,
            pltpu.SemaphoreType.DMA,
            pltpu.SemaphoreType.DMA,
            pltpu.SemaphoreType.DMA,
            pltpu.SemaphoreType.DMA,
            pltpu.SemaphoreType.DMA,
        ],
        compiler_params=pltpu.CompilerParams(
            use_tc_tiling_on_sc=False, needs_layout_passes=False),
    )
    def gather_kernel(table_hbm, idxt_hbm, out_hbm,
                      fla, flb, rowsa, rowsb, tba, tbb,
                      si0, si1, sg0, sg1, so0, so1):
        wid = lax.axis_index("s") * NC + lax.axis_index("c")
        s0w = wid * SW
        flat, rows, tbuf = (fla, flb), (rowsa, rowsb), (tba, tbb)
        si, sg, so = (si0, si1), (sg0, sg1), (so0, so1)

        d_lo = lax.iota(jnp.int32, 16)
        d_hi = d_lo + 16

        def start_idx(h):
            b = h % 2
            return pltpu.async_copy(
                idxt_hbm.at[h, pl.ds(s0w, SW)], flat[b], si[b])

        def start_gather(h):
            b = h % 2
            return pltpu.async_copy(table_hbm.at[flat[b]], rows[b], sg[b])

        def transpose(h):
            b = h % 2
            src, dst = rows[b], tbuf[b]

            @plsc.parallel_loop(0, SW, 1, unroll=8)
            def _(r):
                svec = jnp.full((16,), r, jnp.int32)
                plsc.store_scatter(dst, [d_lo, svec], src[r, pl.ds(0, 16)])
                plsc.store_scatter(dst, [d_hi, svec], src[r, pl.ds(16, 16)])

        def start_out(h):
            b = h % 2
            return pltpu.async_copy(
                tbuf[b].at[:, pl.ds(0, SW)],
                out_hbm.at[h, :, pl.ds(s0w, SW)], so[b])

        hi, hg, ho = [None, None], [None, None], [None, None]
        hi[0] = start_idx(0)
        hi[1] = start_idx(1)
        hi[0].wait()
        hg[0] = start_gather(0)

        for h in range(H):
            b = h % 2
            nb = 1 - b
            hg[b].wait()
            if h + 2 < H:
                hi[b] = start_idx(h + 2)
            if h + 1 < H:
                hi[nb].wait()
                hg[nb] = start_gather(h + 1)
            if h >= 2:
                ho[b].wait()
            transpose(h)
            ho[b] = start_out(h)

        ho[(H - 2) % 2].wait()
        ho[(H - 1) % 2].wait()

    return gather_kernel


def kernel(inputs, embeddings):
    S, H = inputs.shape
    V, D = embeddings.shape
    idx_t = jnp.transpose(inputs).astype(jnp.int32)
    out_t = _make_gather(S, H, V, D)(embeddings, idx_t)
    return jnp.transpose(out_t, (2, 0, 1))


# speedup vs baseline: 3.1262x; 1.1953x over previous
import functools

import jax
import jax.numpy as jnp
from jax import lax
from jax.experimental import pallas as pl
from jax.experimental.pallas import tpu as pltpu
from jax.experimental.pallas import tpu_sc as plsc

TPAD = 131


@functools.cache
def _make_gather(S: int, H: int, V: int, D: int):
    info = plsc.get_sparse_core_info()
    NC, NS = info.num_cores, info.num_subcores
    NW = NC * NS
    assert S % NW == 0
    SW = S // NW
    NTI = D // 8
    NTJ = SW // 128
    NT = NTI * NTJ

    mesh = plsc.VectorSubcoreMesh(core_axis_name="c", subcore_axis_name="s")

    @functools.partial(
        pl.kernel,
        mesh=mesh,
        out_type=jax.ShapeDtypeStruct((H, NTI, S // 128, 8, 128), jnp.float32),
        scratch_types=[
            pltpu.VMEM((SW,), jnp.int32),
            pltpu.VMEM((SW,), jnp.int32),
            pltpu.VMEM((SW, D), jnp.float32),
            pltpu.VMEM((SW, D), jnp.float32),
            pltpu.VMEM((NT, 8, TPAD), jnp.float32),
            pltpu.VMEM((NT, 8, TPAD), jnp.float32),
            pltpu.SemaphoreType.DMA,
            pltpu.SemaphoreType.DMA,
            pltpu.SemaphoreType.DMA,
            pltpu.SemaphoreType.DMA,
            pltpu.SemaphoreType.DMA,
            pltpu.SemaphoreType.DMA,
        ],
        compiler_params=pltpu.CompilerParams(
            use_tc_tiling_on_sc=False, needs_layout_passes=False),
    )
    def gather_kernel(table_hbm, idxt_hbm, out_hbm,
                      fla, flb, rowsa, rowsb, tba, tbb,
                      si0, si1, sg0, sg1, so0, so1):
        wid = lax.axis_index("s") * NC + lax.axis_index("c")
        s0w = wid * SW
        tj0 = wid * NTJ
        flat, rows, tbuf = (fla, flb), (rowsa, rowsb), (tba, tbb)
        si, sg, so = (si0, si1), (sg0, sg1), (so0, so1)

        lane = lax.iota(jnp.int32, 16)
        t_lo = lane >> 3
        t_hi = t_lo + 2
        d_in = lane & 7

        def start_idx(h, b):
            pltpu.async_copy(idxt_hbm.at[h, pl.ds(s0w, SW)], flat[b], si[b])

        def wait_idx(b):
            pltpu.make_async_copy(
                idxt_hbm.at[0, pl.ds(s0w, SW)], flat[b], si[b]).wait()

        def start_gather(b):
            pltpu.async_copy(table_hbm.at[flat[b]], rows[b], sg[b])

        def wait_gather(b):
            pltpu.make_async_copy(
                table_hbm.at[pl.ds(0, SW), :], rows[b], sg[b]).wait()

        def transpose(b):
            src, dst = rows[b], tbuf[b]

            @plsc.parallel_loop(0, SW, 1, unroll=8)
            def _(r):
                tj4 = (r >> 7) << 2
                sv = jnp.full((16,), r & 127, jnp.int32)
                plsc.store_scatter(
                    dst, [t_lo + tj4, d_in, sv], src[r, pl.ds(0, 16)])
                plsc.store_scatter(
                    dst, [t_hi + tj4, d_in, sv], src[r, pl.ds(16, 16)])

        def start_out(h, b):
            for tj in range(NTJ):
                for ti in range(NTI):
                    pltpu.async_copy(
                        tbuf[b].at[tj * NTI + ti, :, pl.ds(0, 128)],
                        out_hbm.at[h, ti, tj0 + tj], so[b])

        def wait_out(b):
            pltpu.make_async_copy(
                table_hbm.at[pl.ds(0, SW), :], rows[b], so[b]).wait()

        def step(h, b, nb, *, start_next_idx, start_next_gather, drain_out):
            wait_gather(b)
            if drain_out:
                wait_out(b)
            if start_next_idx:
                start_idx(h + 2, b)
            if start_next_gather:
                wait_idx(nb)
                start_gather(nb)
            transpose(b)
            start_out(h, b)

        start_idx(0, 0)
        start_idx(1, 1)
        wait_idx(0)
        start_gather(0)
        step(0, 0, 1, start_next_idx=True, start_next_gather=True,
             drain_out=False)
        step(1, 1, 0, start_next_idx=True, start_next_gather=True,
             drain_out=False)

        def body(i, carry):
            h0 = 2 + 2 * i
            step(h0, 0, 1, start_next_idx=True, start_next_gather=True,
                 drain_out=True)
            step(h0 + 1, 1, 0, start_next_idx=True, start_next_gather=True,
                 drain_out=True)
            return carry

        assert H % 2 == 0 and H >= 6
        lax.fori_loop(0, (H - 4) // 2, body, 0)

        step(H - 2, 0, 1, start_next_idx=False, start_next_gather=True,
             drain_out=True)
        step(H - 1, 1, 0, start_next_idx=False, start_next_gather=False,
             drain_out=True)
        wait_out(0)
        wait_out(1)

    return gather_kernel


def kernel(inputs, embeddings):
    S, H = inputs.shape
    V, D = embeddings.shape
    idx_t = jnp.transpose(inputs).astype(jnp.int32)
    out5 = _make_gather(S, H, V, D)(embeddings, idx_t)
    return jnp.transpose(out5, (2, 4, 0, 1, 3)).reshape(S, H, D)
